# baseline (speedup 1.0000x reference)
import functools
import math

import jax
import jax.numpy as jnp
from jax import lax
from jax.experimental import pallas as pl
from jax.experimental.pallas import tpu as pltpu
from jax.experimental.pallas import tpu_sc as plsc

N = 10000
E = 320000
DIM = 128
TIME_DIM = 16
ATOM_DIM = 12
OUT_DIM = 15
N_LAYER = 2
CUTOFF_G = 5.0
RB = 16
HID = DIM + ATOM_DIM + TIME_DIM
HP = 160

NC = 2
NS = 16
L = 16

K = 32
E_PAD = 321536
CPT = E_PAD // (NS * K)
EW = E_PAD // (NC * NS)
N_SP = 10240
SP_STRIPE = N_SP // NS


def _silu(x):
    return x * jax.nn.sigmoid(x)


def _sc_dist_body(px_h, py_h, pz_h, j_h, i_h, s_h, px, py, pz, jv, iv, sv):
    cid = lax.axis_index("c")
    sid = lax.axis_index("s")
    wid = sid * NC + cid
    base = wid * EW
    pltpu.sync_copy(px_h, px)
    pltpu.sync_copy(py_h, py)
    pltpu.sync_copy(pz_h, pz)
    pltpu.sync_copy(j_h.at[pl.ds(base, EW)], jv)
    pltpu.sync_copy(i_h.at[pl.ds(base, EW)], iv)

    def body(k, carry):
        o = k * L
        jj = jv[pl.ds(o, L)]
        ii = iv[pl.ds(o, L)]
        dx = plsc.load_gather(px, [ii]) - plsc.load_gather(px, [jj])
        dy = plsc.load_gather(py, [ii]) - plsc.load_gather(py, [jj])
        dz = plsc.load_gather(pz, [ii]) - plsc.load_gather(pz, [jj])
        sv[pl.ds(o, L)] = dx * dx + dy * dy + dz * dz
        return carry

    lax.fori_loop(0, EW // L, body, 0, unroll=4)
    pltpu.sync_copy(sv, s_h.at[pl.ds(base, EW)])


@functools.partial(jax.jit)
def _sc_dist(px, py, pz, jp, ip):
    mesh = plsc.VectorSubcoreMesh(
        core_axis_name="c", subcore_axis_name="s", num_cores=NC, num_subcores=NS
    )
    return pl.kernel(
        _sc_dist_body,
        out_type=jax.ShapeDtypeStruct((E_PAD,), jnp.float32),
        mesh=mesh,
        compiler_params=pltpu.CompilerParams(needs_layout_passes=False, use_tc_tiling_on_sc=False, internal_scratch_in_bytes=65536),
        scratch_types=[
            pltpu.VMEM((N,), jnp.float32),
            pltpu.VMEM((N,), jnp.float32),
            pltpu.VMEM((N,), jnp.float32),
            pltpu.VMEM((EW,), jnp.int32),
            pltpu.VMEM((EW,), jnp.int32),
            pltpu.VMEM((EW,), jnp.float32),
        ],
    )(px, py, pz, jp, ip)


def _sc_edge_body(shw_h, rbf_h, jiia_h, z_h, agg_h,
                  ji0, ji1, ji2, ji3, ii0, ii1, ii2, ii3,
                  rows0, rows1, rbfv0, rbfv1, agg_sh,
                  sj0, sj1, sj2, sj3, si0, si1, si2, si3,
                  g0, g1, r0, r1, s0, s1):
    cid = lax.axis_index("c")
    sid = lax.axis_index("s")
    ji = [ji0, ji1, ji2, ji3]
    ii = [ii0, ii1, ii2, ii3]
    sj = [sj0, sj1, sj2, sj3]
    si = [si0, si1, si2, si3]
    rows = [rows0, rows1]
    rbfv = [rbfv0, rbfv1]
    g = [g0, g1]
    r = [r0, r1]
    s = [s0, s1]
    cbase = sid * CPT
    cmax = CPT - 1

    def issue_idx(c, xs):
        c = jnp.minimum(c, cmax)
        pltpu.async_copy(jiia_h.at[cid, 0, pl.ds((cbase + c) * K, K)],
                         ji[xs], sj[xs])
        pltpu.async_copy(jiia_h.at[cid, 1, pl.ds((cbase + c) * K, K)],
                         ii[xs], si[xs])

    def issue_gr(c, rs, xs):
        c = jnp.minimum(c, cmax)
        pltpu.async_copy(shw_h.at[ji[xs]], rows[rs], g[rs])
        pltpu.async_copy(rbf_h.at[pl.ds((cbase + c) * K, K), :],
                         rbfv[rs], r[rs])

    def wait_rows(sem, rowbuf):
        pltpu.make_async_copy(z_h.at[pl.ds(0, K), :], rowbuf, sem).wait()

    def wait_rbf(sem, rbfbuf):
        pltpu.make_async_copy(rbf_h.at[pl.ds(0, K), :], rbfbuf, sem).wait()

    def wait_idx(sem, ibuf):
        pltpu.make_async_copy(jiia_h.at[cid, 0, pl.ds(0, K)], ibuf,
                              sem).wait()

    def mul(rs):
        def mulrow(rw, carry2):
            for q in range(HP // (2 * L)):
                lo, hi = plsc.unpack(rbfv[rs][rw, pl.ds(q * 2 * L, 2 * L)],
                                     format=plsc.PackFormat.INTERLEAVED)
                rows[rs][rw, pl.ds(q * 2 * L, L)] = (
                    rows[rs][rw, pl.ds(q * 2 * L, L)] * lo
                )
                rows[rs][rw, pl.ds(q * 2 * L + L, L)] = (
                    rows[rs][rw, pl.ds(q * 2 * L + L, L)] * hi
                )
            return carry2

        lax.fori_loop(0, K, mulrow, 0, unroll=16)

    pltpu.sync_copy(z_h.at[pl.ds(sid * SP_STRIPE, SP_STRIPE), :],
                    agg_sh.at[pl.ds(sid * SP_STRIPE, SP_STRIPE), :])
    plsc.subcore_barrier()

    issue_idx(0, 0)
    issue_idx(1, 1)
    issue_idx(2, 2)
    wait_idx(sj[0], ji[0])
    issue_gr(0, 0, 0)
    pltpu.async_copy(z_h.at[pl.ds(0, K), :], rows[1], s[1])

    def quad(it, carry):
        for u in range(4):
            c = it * 4 + u
            rs, xs = u % 2, u
            nrs, nxs = 1 - rs, (u + 1) % 4
            wait_rows(s[nrs], rows[nrs])
            wait_idx(sj[nxs], ji[nxs])
            issue_gr(c + 1, nrs, nxs)
            issue_idx(c + 3, (u + 3) % 4)
            wait_rows(g[rs], rows[rs])
            wait_rbf(r[rs], rbfv[rs])
            mul(rs)
            wait_idx(si[xs], ii[xs])
            pltpu.async_copy(rows[rs], agg_sh.at[ii[xs]], s[rs], add=True)
        return carry

    lax.fori_loop(0, CPT // 4, quad, 0)
    wait_rows(s[1], rows[1])
    wait_rows(g[0], rows[0])
    wait_rbf(r[0], rbfv[0])
    wait_idx(sj[1], ji[1])
    wait_idx(sj[2], ji[2])
    wait_idx(si[0], ii[0])
    wait_idx(si[1], ii[1])
    wait_idx(si[2], ii[2])
    plsc.subcore_barrier()
    pltpu.sync_copy(
        agg_sh.at[pl.ds(sid * SP_STRIPE, SP_STRIPE), :],
        agg_h.at[pl.ds(cid * N_SP + sid * SP_STRIPE, SP_STRIPE), :],
    )


def _sc_edge(shw2, rbf, jiia, zer):
    mesh = plsc.VectorSubcoreMesh(
        core_axis_name="c", subcore_axis_name="s", num_cores=NC, num_subcores=NS
    )
    return pl.kernel(
        _sc_edge_body,
        out_type=jax.ShapeDtypeStruct((2 * N_SP, HP), jnp.float32),
        mesh=mesh,
        compiler_params=pltpu.CompilerParams(needs_layout_passes=False, use_tc_tiling_on_sc=False),
        scratch_types=(
            [pltpu.VMEM((K,), jnp.int32)] * 8
            + [pltpu.VMEM((K, HP), jnp.float32)] * 2
            + [pltpu.VMEM((K, HP), jnp.bfloat16)] * 2
            + [pltpu.VMEM_SHARED((N_SP, HP), jnp.float32)]
            + [pltpu.SemaphoreType.DMA] * 14
        ),
    )(shw2, rbf, jiia, zer)


BER = 16
BE = BER * 128


def _tc_rbf_body(s_ref, ea_ref, w_ref, b_ref, o_ref):
    p = 5
    a = -(p + 1) * (p + 2) / 2.0
    b = p * (p + 2)
    c = -p * (p + 1) / 2.0
    s = s_ref[...]
    dist = jnp.sqrt(s + 1e-12)
    d = jnp.clip(dist / CUTOFF_G, 1e-3, 1.0)
    d2 = d * d
    d4 = d2 * d2
    env = 1.0 / d + a * d4 + b * d4 * d + c * d4 * d2
    kcol = (lax.broadcasted_iota(jnp.int32, (RB, 1), 0).astype(jnp.float32)
            + 1.0) * math.pi
    ea = ea_ref[...]
    w = w_ref[...]
    brow = b_ref[...]
    eblk = pl.program_id(0) * BE
    for u in range(BER):
        d_u = lax.slice(d, (u, 0), (u + 1, 128))
        env_u = lax.slice(env, (u, 0), (u + 1, 128))
        sin_u = jnp.sin(kcol * d_u) * env_u
        ea_u = lax.slice(ea, (0, u, 0), (3, u + 1, 128)).reshape(3, 128)
        cat = jnp.concatenate([sin_u, ea_u], axis=0)
        z = lax.dot_general(cat, w, (((0,), (0,)), ((), ())),
                            preferred_element_type=jnp.float32) + brow
        val = _silu(z)
        val = jnp.where(eblk + u * 128 < E, val, 0.0)
        o_ref[pl.ds(u * 128, 128), :] = val.astype(jnp.bfloat16)


def _tc_rbf(sR, eaT, w, brow):
    return pl.pallas_call(
        _tc_rbf_body,
        grid=(E_PAD // BE,),
        in_specs=[
            pl.BlockSpec((BER, 128), lambda e: (e, 0)),
            pl.BlockSpec((3, BER, 128), lambda e: (0, e, 0)),
            pl.BlockSpec((RB + 3, HP), lambda e: (0, 0)),
            pl.BlockSpec((1, HP), lambda e: (0, 0)),
        ],
        out_specs=pl.BlockSpec((BE, HP), lambda e: (e, 0)),
        out_shape=jax.ShapeDtypeStruct((E_PAD, HP), jnp.bfloat16),
    )(sR, eaT, w, brow)


BN = 1000
NB = N // BN


def _tc_shw0_body(x_ref, wm_ref, o_ref):
    x = x_ref[0]
    o_ref[0] = _silu(jnp.dot(x, wm_ref[0], preferred_element_type=jnp.float32))


def _tc_shw0(x, wm0):
    return pl.pallas_call(
        _tc_shw0_body,
        grid=(2, NB),
        in_specs=[
            pl.BlockSpec((1, BN, HID), lambda s, n: (0, n, 0)),
            pl.BlockSpec((1, HID, HP), lambda s, n: (s, 0, 0)),
        ],
        out_specs=pl.BlockSpec((1, BN, HP), lambda s, n: (s, n, 0)),
        out_shape=jax.ShapeDtypeStruct((2, N_SP, HP), jnp.float32),
    )(x, wm0)


def _tc_dense_body_shw(h_ref, agg_ref, wu_ref, wo_ref, wm_ref, oin_ref,
                       h_o, out_o, shw_o):
    h = h_ref[0]
    agg = agg_ref[0][:, :HID]
    hn = _silu(jnp.dot(h + agg, wu_ref[0], preferred_element_type=jnp.float32))
    h_o[0] = hn
    out_o[0] = oin_ref[0] + jnp.dot(hn, wo_ref[0],
                                    preferred_element_type=jnp.float32)
    shw_o[0] = _silu(jnp.dot(hn, wm_ref[0], preferred_element_type=jnp.float32))


def _tc_dense_body(h_ref, agg_ref, wu_ref, wo_ref, oin_ref, h_o, out_o):
    h = h_ref[0]
    agg = agg_ref[0][:, :HID]
    hn = _silu(jnp.dot(h + agg, wu_ref[0], preferred_element_type=jnp.float32))
    h_o[0] = hn
    out_o[0] = oin_ref[0] + jnp.dot(hn, wo_ref[0],
                                    preferred_element_type=jnp.float32)


def _tc_dense(h, agg, wu, wo, oin, wm_next=None, share_h=False):
    in_specs = [
        pl.BlockSpec((1, BN, HID),
                     (lambda s, n: (0, n, 0)) if share_h
                     else (lambda s, n: (s, n, 0))),
        pl.BlockSpec((1, BN, HP), lambda s, n: (s, n, 0)),
        pl.BlockSpec((1, HID, HID), lambda s, n: (s, 0, 0)),
        pl.BlockSpec((1, HID, OUT_DIM), lambda s, n: (s, 0, 0)),
    ]
    out_specs = [
        pl.BlockSpec((1, BN, HID), lambda s, n: (s, n, 0)),
        pl.BlockSpec((1, BN, OUT_DIM), lambda s, n: (s, n, 0)),
    ]
    out_shape = [
        jax.ShapeDtypeStruct((2, N, HID), jnp.float32),
        jax.ShapeDtypeStruct((2, N, OUT_DIM), jnp.float32),
    ]
    oin_spec = pl.BlockSpec((1, BN, OUT_DIM), lambda s, n: (s, n, 0))
    if wm_next is not None:
        return pl.pallas_call(
            _tc_dense_body_shw,
            grid=(2, NB),
            in_specs=in_specs
            + [pl.BlockSpec((1, HID, HP), lambda s, n: (s, 0, 0)), oin_spec],
            out_specs=out_specs
            + [pl.BlockSpec((1, BN, HP), lambda s, n: (s, n, 0))],
            out_shape=out_shape
            + [jax.ShapeDtypeStruct((2, N_SP, HP), jnp.float32)],
        )(h, agg, wu, wo, wm_next, oin)
    return pl.pallas_call(
        _tc_dense_body,
        grid=(2, NB),
        in_specs=in_specs + [oin_spec],
        out_specs=out_specs,
        out_shape=out_shape,
    )(h, agg, wu, wo, oin)


def _tc_final_body(og_ref, ol_ref, te_ref, w_ref, b_ref, o_ref):
    w = w_ref[...]
    val = (jnp.dot(og_ref[...], w[:OUT_DIM], preferred_element_type=jnp.float32)
           + jnp.dot(ol_ref[...], w[OUT_DIM:2 * OUT_DIM],
                     preferred_element_type=jnp.float32)
           + jnp.dot(te_ref[...], w[2 * OUT_DIM:],
                     preferred_element_type=jnp.float32)
           + b_ref[...])
    o_ref[...] = val


def _tc_final(og, ol, te, w, brow):
    return pl.pallas_call(
        _tc_final_body,
        grid=(NB,),
        in_specs=[
            pl.BlockSpec((BN, OUT_DIM), lambda n: (n, 0)),
            pl.BlockSpec((BN, OUT_DIM), lambda n: (n, 0)),
            pl.BlockSpec((BN, TIME_DIM), lambda n: (n, 0)),
            pl.BlockSpec((2 * OUT_DIM + TIME_DIM, OUT_DIM), lambda n: (0, 0)),
            pl.BlockSpec((1, OUT_DIM), lambda n: (0, 0)),
        ],
        out_specs=pl.BlockSpec((BN, OUT_DIM), lambda n: (n, 0)),
        out_shape=jax.ShapeDtypeStruct((N, OUT_DIM), jnp.float32),
    )(og, ol, te, w, brow)


def _sinusoidal_emb(time, dim):
    half = dim // 2
    f = math.log(10000.0) / (half - 1)
    freqs = jnp.exp(jnp.arange(half, dtype=jnp.float32) * -f)
    e = time[:, None] * freqs[None, :]
    return jnp.concatenate([jnp.sin(e), jnp.cos(e)], axis=-1)


def kernel(x_raw, edge_index, edge_attr, t, W_init, b_init, Wt1, bt1, Wt2, bt2,
           Wrbf, brbf, Wm_g, Wu_g, Wo_g, Wm_l, Wu_l, Wo_l, Wout, bout):
    pos = x_raw[:, :3]
    feats = x_raw[:, 3:]
    x_pos = _silu(pos @ W_init + b_init)
    temb = _sinusoidal_emb(t, DIM)
    time_emb = jax.nn.gelu(temb @ Wt1 + bt1) @ Wt2 + bt2
    x = jnp.concatenate([x_pos, feats, time_emb], axis=1)

    jp = jnp.pad(edge_index[0], (0, E_PAD - E))
    ip = jnp.pad(edge_index[1], (0, E_PAD - E))
    px = pos[:, 0]
    py = pos[:, 1]
    pz = pos[:, 2]

    s = _sc_dist(px, py, pz, jp, ip)
    sR = s.reshape(E_PAD // 128, 128)
    eaT = jnp.pad(edge_attr, ((0, E_PAD - E), (0, 0))).T.reshape(
        3, E_PAD // 128, 128)
    wr = jnp.pad(Wrbf, ((0, 0), (0, HP - HID)))
    br = jnp.pad(brbf, (0, HP - HID)).reshape(1, HP)
    q32 = jnp.arange(HP) // 32
    t16 = (jnp.arange(HP) % 32) // 2
    odd = jnp.arange(HP) % 2
    perm = q32 * 32 + t16 + odd * 16
    rbf_feat = _tc_rbf(sR, eaT, wr[:, perm], br[:, perm])
    jiia = jnp.stack([jnp.stack([jp, ip]), jnp.stack([jp + N_SP, ip])])

    wm = jnp.stack([Wm_g, Wm_l])
    wm = jnp.pad(wm, ((0, 0), (0, 0), (0, 0), (0, HP - HID)))
    wu = jnp.stack([Wu_g, Wu_l])
    wo = jnp.stack([Wo_g, Wo_l])

    zer = jnp.zeros((N_SP, HP), jnp.float32)

    shw0 = _tc_shw0(x.reshape(1, N, HID), wm[:, 0])
    agg0 = _sc_edge(shw0.reshape(2 * N_SP, HP), rbf_feat, jiia, zer)
    oin = jnp.zeros((2, N, OUT_DIM), jnp.float32)
    h1, out1, shw1 = _tc_dense(x.reshape(1, N, HID), agg0.reshape(2, N_SP, HP),
                               wu[:, 0], wo[:, 0], oin, wm_next=wm[:, 1],
                               share_h=True)
    agg1 = _sc_edge(shw1.reshape(2 * N_SP, HP), rbf_feat, jiia, zer)
    h2, out2 = _tc_dense(h1, agg1.reshape(2, N_SP, HP), wu[:, 1], wo[:, 1],
                         out1)
    final = _tc_final(out2[0], out2[1], time_emb,
                      Wout, bout.reshape(1, OUT_DIM))
    return final


# speedup vs baseline: 1.5341x; 1.5341x over previous
import functools
import math

import jax
import jax.numpy as jnp
from jax import lax
from jax.experimental import pallas as pl
from jax.experimental.pallas import tpu as pltpu
from jax.experimental.pallas import tpu_sc as plsc

N = 10000
E = 320000
DIM = 128
TIME_DIM = 16
ATOM_DIM = 12
OUT_DIM = 15
N_LAYER = 2
CUTOFF_G = 5.0
RB = 16
HID = DIM + ATOM_DIM + TIME_DIM
HP = 160

NC = 2
NS = 16
L = 16

K = 32
E_PAD = 321536
CPT = E_PAD // (NS * K)
EW = E_PAD // (NC * NS)
N_SP = 10240
SP_STRIPE = N_SP // NS


def _silu(x):
    return x * jax.nn.sigmoid(x)


def _sc_dist_body(px_h, py_h, pz_h, j_h, i_h, s_h, px, py, pz, jv, iv, sv):
    cid = lax.axis_index("c")
    sid = lax.axis_index("s")
    wid = sid * NC + cid
    base = wid * EW
    pltpu.sync_copy(px_h, px)
    pltpu.sync_copy(py_h, py)
    pltpu.sync_copy(pz_h, pz)
    pltpu.sync_copy(j_h.at[pl.ds(base, EW)], jv)
    pltpu.sync_copy(i_h.at[pl.ds(base, EW)], iv)

    def body(k, carry):
        o = k * L
        jj = jv[pl.ds(o, L)]
        ii = iv[pl.ds(o, L)]
        dx = plsc.load_gather(px, [ii]) - plsc.load_gather(px, [jj])
        dy = plsc.load_gather(py, [ii]) - plsc.load_gather(py, [jj])
        dz = plsc.load_gather(pz, [ii]) - plsc.load_gather(pz, [jj])
        sv[pl.ds(o, L)] = dx * dx + dy * dy + dz * dz
        return carry

    lax.fori_loop(0, EW // L, body, 0, unroll=4)
    pltpu.sync_copy(sv, s_h.at[pl.ds(base, EW)])


@functools.partial(jax.jit)
def _sc_dist(px, py, pz, jp, ip):
    mesh = plsc.VectorSubcoreMesh(
        core_axis_name="c", subcore_axis_name="s", num_cores=NC, num_subcores=NS
    )
    return pl.kernel(
        _sc_dist_body,
        out_type=jax.ShapeDtypeStruct((E_PAD,), jnp.float32),
        mesh=mesh,
        compiler_params=pltpu.CompilerParams(needs_layout_passes=False, use_tc_tiling_on_sc=False, internal_scratch_in_bytes=65536),
        scratch_types=[
            pltpu.VMEM((N,), jnp.float32),
            pltpu.VMEM((N,), jnp.float32),
            pltpu.VMEM((N,), jnp.float32),
            pltpu.VMEM((EW,), jnp.int32),
            pltpu.VMEM((EW,), jnp.int32),
            pltpu.VMEM((EW,), jnp.float32),
        ],
    )(px, py, pz, jp, ip)


def _sc_edge_body(shw_h, rbf_h, jiia_h, z_h, agg_h,
                  ji0, ji1, ji2, ji3, ii0, ii1, ii2, ii3,
                  rows0, rows1, rbfv0, rbfv1, agg_sh,
                  sj0, sj1, sj2, sj3, si0, si1, si2, si3,
                  g0, g1, r0, r1, s0, s1):
    cid = lax.axis_index("c")
    sid = lax.axis_index("s")
    ji = [ji0, ji1, ji2, ji3]
    ii = [ii0, ii1, ii2, ii3]
    sj = [sj0, sj1, sj2, sj3]
    si = [si0, si1, si2, si3]
    rows = [rows0, rows1]
    rbfv = [rbfv0, rbfv1]
    g = [g0, g1]
    r = [r0, r1]
    s = [s0, s1]
    cbase = sid * CPT
    cmax = CPT - 1

    def issue_idx(c, xs):
        c = jnp.minimum(c, cmax)
        pltpu.async_copy(jiia_h.at[cid, 0, pl.ds((cbase + c) * K, K)],
                         ji[xs], sj[xs])
        pltpu.async_copy(jiia_h.at[cid, 1, pl.ds((cbase + c) * K, K)],
                         ii[xs], si[xs])

    def issue_gr(c, rs, xs):
        c = jnp.minimum(c, cmax)
        pltpu.async_copy(shw_h.at[ji[xs]], rows[rs], g[rs])
        pltpu.async_copy(rbf_h.at[pl.ds((cbase + c) * K, K), :],
                         rbfv[rs], r[rs])

    def wait_rows(sem, rowbuf):
        pltpu.make_async_copy(z_h.at[pl.ds(0, K), :], rowbuf, sem).wait()

    def wait_idx(sem, ibuf):
        pltpu.make_async_copy(jiia_h.at[cid, 0, pl.ds(0, K)], ibuf,
                              sem).wait()

    def mul(rs):
        def mulrow(rw, carry2):
            for q in range(HP // L):
                rows[rs][rw, pl.ds(q * L, L)] = (
                    rows[rs][rw, pl.ds(q * L, L)]
                    * rbfv[rs][rw, pl.ds(q * L, L)]
                )
            return carry2

        lax.fori_loop(0, K, mulrow, 0, unroll=16)

    pltpu.sync_copy(z_h.at[pl.ds(sid * SP_STRIPE, SP_STRIPE), :],
                    agg_sh.at[pl.ds(sid * SP_STRIPE, SP_STRIPE), :])
    plsc.subcore_barrier()

    issue_idx(0, 0)
    issue_idx(1, 1)
    issue_idx(2, 2)
    wait_idx(sj[0], ji[0])
    issue_gr(0, 0, 0)
    pltpu.async_copy(z_h.at[pl.ds(0, K), :], rows[1], s[1])

    def quad(it, carry):
        for u in range(4):
            c = it * 4 + u
            rs, xs = u % 2, u
            nrs, nxs = 1 - rs, (u + 1) % 4
            wait_rows(s[nrs], rows[nrs])
            wait_idx(sj[nxs], ji[nxs])
            issue_gr(c + 1, nrs, nxs)
            issue_idx(c + 3, (u + 3) % 4)
            wait_rows(g[rs], rows[rs])
            wait_rows(r[rs], rbfv[rs])
            mul(rs)
            wait_idx(si[xs], ii[xs])
            pltpu.async_copy(rows[rs], agg_sh.at[ii[xs]], s[rs], add=True)
        return carry

    lax.fori_loop(0, CPT // 4, quad, 0)
    wait_rows(s[1], rows[1])
    wait_rows(g[0], rows[0])
    wait_rows(r[0], rbfv[0])
    wait_idx(sj[1], ji[1])
    wait_idx(sj[2], ji[2])
    wait_idx(si[0], ii[0])
    wait_idx(si[1], ii[1])
    wait_idx(si[2], ii[2])
    plsc.subcore_barrier()
    pltpu.sync_copy(
        agg_sh.at[pl.ds(sid * SP_STRIPE, SP_STRIPE), :],
        agg_h.at[pl.ds(cid * N_SP + sid * SP_STRIPE, SP_STRIPE), :],
    )


def _sc_edge(shw2, rbf, jiia, zer):
    mesh = plsc.VectorSubcoreMesh(
        core_axis_name="c", subcore_axis_name="s", num_cores=NC, num_subcores=NS
    )
    return pl.kernel(
        _sc_edge_body,
        out_type=jax.ShapeDtypeStruct((2 * N_SP, HP), jnp.float32),
        mesh=mesh,
        compiler_params=pltpu.CompilerParams(needs_layout_passes=False, use_tc_tiling_on_sc=False),
        scratch_types=(
            [pltpu.VMEM((K,), jnp.int32)] * 8
            + [pltpu.VMEM((K, HP), jnp.float32)] * 4
            + [pltpu.VMEM_SHARED((N_SP, HP), jnp.float32)]
            + [pltpu.SemaphoreType.DMA] * 14
        ),
    )(shw2, rbf, jiia, zer)


BER = 16
BE = BER * 128


def _tc_rbf_body(s_ref, ea_ref, w_ref, b_ref, o_ref):
    p = 5
    a = -(p + 1) * (p + 2) / 2.0
    b = p * (p + 2)
    c = -p * (p + 1) / 2.0
    s = s_ref[...]
    dist = jnp.sqrt(s + 1e-12)
    d = jnp.clip(dist / CUTOFF_G, 1e-3, 1.0)
    d2 = d * d
    d4 = d2 * d2
    env = 1.0 / d + a * d4 + b * d4 * d + c * d4 * d2
    kcol = (lax.broadcasted_iota(jnp.int32, (RB, 1), 0).astype(jnp.float32)
            + 1.0) * math.pi
    ea = ea_ref[...]
    w = w_ref[...]
    brow = b_ref[...]
    eblk = pl.program_id(0) * BE
    for u in range(BER):
        d_u = lax.slice(d, (u, 0), (u + 1, 128))
        env_u = lax.slice(env, (u, 0), (u + 1, 128))
        sin_u = jnp.sin(kcol * d_u) * env_u
        ea_u = lax.slice(ea, (0, u, 0), (3, u + 1, 128)).reshape(3, 128)
        cat = jnp.concatenate([sin_u, ea_u], axis=0)
        z = lax.dot_general(cat, w, (((0,), (0,)), ((), ())),
                            preferred_element_type=jnp.float32) + brow
        val = _silu(z)
        val = jnp.where(eblk + u * 128 < E, val, 0.0)
        o_ref[pl.ds(u * 128, 128), :] = val


def _tc_rbf(sR, eaT, w, brow):
    return pl.pallas_call(
        _tc_rbf_body,
        grid=(E_PAD // BE,),
        in_specs=[
            pl.BlockSpec((BER, 128), lambda e: (e, 0)),
            pl.BlockSpec((3, BER, 128), lambda e: (0, e, 0)),
            pl.BlockSpec((RB + 3, HP), lambda e: (0, 0)),
            pl.BlockSpec((1, HP), lambda e: (0, 0)),
        ],
        out_specs=pl.BlockSpec((BE, HP), lambda e: (e, 0)),
        out_shape=jax.ShapeDtypeStruct((E_PAD, HP), jnp.float32),
    )(sR, eaT, w, brow)


BN = 1000
NB = N // BN


def _tc_shw0_body(x_ref, wm_ref, o_ref):
    x = x_ref[0]
    o_ref[0] = _silu(jnp.dot(x, wm_ref[0], preferred_element_type=jnp.float32))


def _tc_shw0(x, wm0):
    return pl.pallas_call(
        _tc_shw0_body,
        grid=(2, NB),
        in_specs=[
            pl.BlockSpec((1, BN, HID), lambda s, n: (0, n, 0)),
            pl.BlockSpec((1, HID, HP), lambda s, n: (s, 0, 0)),
        ],
        out_specs=pl.BlockSpec((1, BN, HP), lambda s, n: (s, n, 0)),
        out_shape=jax.ShapeDtypeStruct((2, N_SP, HP), jnp.float32),
    )(x, wm0)


def _tc_dense_body_shw(h_ref, agg_ref, wu_ref, wo_ref, wm_ref, oin_ref,
                       h_o, out_o, shw_o):
    h = h_ref[0]
    agg = agg_ref[0][:, :HID]
    hn = _silu(jnp.dot(h + agg, wu_ref[0], preferred_element_type=jnp.float32))
    h_o[0] = hn
    out_o[0] = oin_ref[0] + jnp.dot(hn, wo_ref[0],
                                    preferred_element_type=jnp.float32)
    shw_o[0] = _silu(jnp.dot(hn, wm_ref[0], preferred_element_type=jnp.float32))


def _tc_dense_body(h_ref, agg_ref, wu_ref, wo_ref, oin_ref, h_o, out_o):
    h = h_ref[0]
    agg = agg_ref[0][:, :HID]
    hn = _silu(jnp.dot(h + agg, wu_ref[0], preferred_element_type=jnp.float32))
    h_o[0] = hn
    out_o[0] = oin_ref[0] + jnp.dot(hn, wo_ref[0],
                                    preferred_element_type=jnp.float32)


def _tc_dense(h, agg, wu, wo, oin, wm_next=None, share_h=False):
    in_specs = [
        pl.BlockSpec((1, BN, HID),
                     (lambda s, n: (0, n, 0)) if share_h
                     else (lambda s, n: (s, n, 0))),
        pl.BlockSpec((1, BN, HP), lambda s, n: (s, n, 0)),
        pl.BlockSpec((1, HID, HID), lambda s, n: (s, 0, 0)),
        pl.BlockSpec((1, HID, OUT_DIM), lambda s, n: (s, 0, 0)),
    ]
    out_specs = [
        pl.BlockSpec((1, BN, HID), lambda s, n: (s, n, 0)),
        pl.BlockSpec((1, BN, OUT_DIM), lambda s, n: (s, n, 0)),
    ]
    out_shape = [
        jax.ShapeDtypeStruct((2, N, HID), jnp.float32),
        jax.ShapeDtypeStruct((2, N, OUT_DIM), jnp.float32),
    ]
    oin_spec = pl.BlockSpec((1, BN, OUT_DIM), lambda s, n: (s, n, 0))
    if wm_next is not None:
        return pl.pallas_call(
            _tc_dense_body_shw,
            grid=(2, NB),
            in_specs=in_specs
            + [pl.BlockSpec((1, HID, HP), lambda s, n: (s, 0, 0)), oin_spec],
            out_specs=out_specs
            + [pl.BlockSpec((1, BN, HP), lambda s, n: (s, n, 0))],
            out_shape=out_shape
            + [jax.ShapeDtypeStruct((2, N_SP, HP), jnp.float32)],
        )(h, agg, wu, wo, wm_next, oin)
    return pl.pallas_call(
        _tc_dense_body,
        grid=(2, NB),
        in_specs=in_specs + [oin_spec],
        out_specs=out_specs,
        out_shape=out_shape,
    )(h, agg, wu, wo, oin)


def _tc_final_body(og_ref, ol_ref, te_ref, w_ref, b_ref, o_ref):
    w = w_ref[...]
    val = (jnp.dot(og_ref[...], w[:OUT_DIM], preferred_element_type=jnp.float32)
           + jnp.dot(ol_ref[...], w[OUT_DIM:2 * OUT_DIM],
                     preferred_element_type=jnp.float32)
           + jnp.dot(te_ref[...], w[2 * OUT_DIM:],
                     preferred_element_type=jnp.float32)
           + b_ref[...])
    o_ref[...] = val


def _tc_final(og, ol, te, w, brow):
    return pl.pallas_call(
        _tc_final_body,
        grid=(NB,),
        in_specs=[
            pl.BlockSpec((BN, OUT_DIM), lambda n: (n, 0)),
            pl.BlockSpec((BN, OUT_DIM), lambda n: (n, 0)),
            pl.BlockSpec((BN, TIME_DIM), lambda n: (n, 0)),
            pl.BlockSpec((2 * OUT_DIM + TIME_DIM, OUT_DIM), lambda n: (0, 0)),
            pl.BlockSpec((1, OUT_DIM), lambda n: (0, 0)),
        ],
        out_specs=pl.BlockSpec((BN, OUT_DIM), lambda n: (n, 0)),
        out_shape=jax.ShapeDtypeStruct((N, OUT_DIM), jnp.float32),
    )(og, ol, te, w, brow)


def _sinusoidal_emb(time, dim):
    half = dim // 2
    f = math.log(10000.0) / (half - 1)
    freqs = jnp.exp(jnp.arange(half, dtype=jnp.float32) * -f)
    e = time[:, None] * freqs[None, :]
    return jnp.concatenate([jnp.sin(e), jnp.cos(e)], axis=-1)


def kernel(x_raw, edge_index, edge_attr, t, W_init, b_init, Wt1, bt1, Wt2, bt2,
           Wrbf, brbf, Wm_g, Wu_g, Wo_g, Wm_l, Wu_l, Wo_l, Wout, bout):
    pos = x_raw[:, :3]
    feats = x_raw[:, 3:]
    x_pos = _silu(pos @ W_init + b_init)
    temb = _sinusoidal_emb(t, DIM)
    time_emb = jax.nn.gelu(temb @ Wt1 + bt1) @ Wt2 + bt2
    x = jnp.concatenate([x_pos, feats, time_emb], axis=1)

    jp = jnp.pad(edge_index[0], (0, E_PAD - E))
    ip = jnp.pad(edge_index[1], (0, E_PAD - E))
    px = pos[:, 0]
    py = pos[:, 1]
    pz = pos[:, 2]

    s = _sc_dist(px, py, pz, jp, ip)
    sR = s.reshape(E_PAD // 128, 128)
    eaT = jnp.pad(edge_attr, ((0, E_PAD - E), (0, 0))).T.reshape(
        3, E_PAD // 128, 128)
    wr = jnp.pad(Wrbf, ((0, 0), (0, HP - HID)))
    rbf_feat = _tc_rbf(sR, eaT, wr,
                       jnp.pad(brbf, (0, HP - HID)).reshape(1, HP))
    jiia = jnp.stack([jnp.stack([jp, ip]), jnp.stack([jp + N_SP, ip])])

    wm = jnp.stack([Wm_g, Wm_l])
    wm = jnp.pad(wm, ((0, 0), (0, 0), (0, 0), (0, HP - HID)))
    wu = jnp.stack([Wu_g, Wu_l])
    wo = jnp.stack([Wo_g, Wo_l])

    zer = jnp.zeros((N_SP, HP), jnp.float32)

    shw0 = _tc_shw0(x.reshape(1, N, HID), wm[:, 0])
    agg0 = _sc_edge(shw0.reshape(2 * N_SP, HP), rbf_feat, jiia, zer)
    oin = jnp.zeros((2, N, OUT_DIM), jnp.float32)
    h1, out1, shw1 = _tc_dense(x.reshape(1, N, HID), agg0.reshape(2, N_SP, HP),
                               wu[:, 0], wo[:, 0], oin, wm_next=wm[:, 1],
                               share_h=True)
    agg1 = _sc_edge(shw1.reshape(2 * N_SP, HP), rbf_feat, jiia, zer)
    h2, out2 = _tc_dense(h1, agg1.reshape(2, N_SP, HP), wu[:, 1], wo[:, 1],
                         out1)
    final = _tc_final(out2[0], out2[1], time_emb,
                      Wout, bout.reshape(1, OUT_DIM))
    return final
